# baseline (speedup 1.0000x reference)
import functools

import jax
import jax.numpy as jnp
from jax import lax
from jax.experimental import pallas as pl
from jax.experimental.pallas import tpu as pltpu
from jax.experimental.pallas import tpu_sc as plsc

_B = 4096
_T = 50
_D = 64
_N = _B * _T
_NE = 1000000
_NR = 1000
_NC = 2
_NS = 16
_NW = _NC * _NS
_BW = _B // _NW

_TXB = 8192


def _tx_body(in_ref, eye_ref, out_ref):
    out_ref[:, 0:_D] = jax.lax.dot_general(
        in_ref[...], eye_ref[...], (((0,), (0,)), ((), ())),
        preferred_element_type=jnp.float32,
        precision=jax.lax.Precision.HIGHEST)


def _stage(tbl_t, rows):
    eye = jnp.eye(_D, dtype=jnp.float32)
    grid = -(-rows // _TXB)
    return pl.pallas_call(
        _tx_body,
        grid=(grid,),
        in_specs=[pl.BlockSpec((_D, _TXB), lambda i: (0, i)),
                  pl.BlockSpec((_D, _D), lambda i: (0, 0))],
        out_specs=pl.BlockSpec((_TXB, 128), lambda i: (i, 0)),
        out_shape=jax.ShapeDtypeStruct((rows, 128), jnp.float32),
    )(tbl_t, eye)


def _transpose_block(buf, tbuf):
    def qbody(q, carry):
        for s in range(8):
            col = jnp.broadcast_to(q * 8 + s, (16,)).astype(jnp.int32)
            for li in range(8):
                rows = lax.iota(jnp.int32, 16) + li * 16
                v = plsc.load_gather(buf, [rows, col])
                tbuf[q, 0, s, pl.ds(li * 16, 16)] = v
        return carry

    lax.fori_loop(0, 8, qbody, 0)


def _gather_body(idx_hbm, table, out_hbm,
                 idx_v, buf_a, buf_b, tbuf_a, tbuf_b, g0, g1):
    wid = lax.axis_index("s") * _NC + lax.axis_index("c")
    b0 = wid * _BW

    pltpu.sync_copy(idx_hbm.at[:, pl.ds(b0, _BW)], idx_v)

    def step(t, buf, tbuf, gsem):
        gd = pltpu.async_copy(table.at[idx_v.at[t]], buf, gsem)
        return gd

    def write(t, tbuf):
        pltpu.sync_copy(
            tbuf, out_hbm.at[pl.ds(t * 8, 8), pl.ds(wid, 1)])

    def body(i, carry):
        t0 = i * 2
        ga = step(t0, buf_a, tbuf_a, g0)
        gb = step(t0 + 1, buf_b, tbuf_b, g1)
        ga.wait()
        _transpose_block(buf_a, tbuf_a)
        write(t0, tbuf_a)
        gb.wait()
        _transpose_block(buf_b, tbuf_b)
        write(t0 + 1, tbuf_b)
        return carry

    lax.fori_loop(0, _T // 2, body, 0)


def _make_gather():
    mesh = plsc.VectorSubcoreMesh(
        core_axis_name="c", subcore_axis_name="s",
        num_cores=_NC, num_subcores=_NS)
    return pl.kernel(
        _gather_body,
        out_type=jax.ShapeDtypeStruct((_T * 8, _NW, 8, 128), jnp.float32),
        mesh=mesh,
        scratch_types=[
            pltpu.VMEM((_T, _BW), jnp.int32),
            pltpu.VMEM((_BW, 128), jnp.float32),
            pltpu.VMEM((_BW, 128), jnp.float32),
            pltpu.VMEM((8, 1, 8, 128), jnp.float32),
            pltpu.VMEM((8, 1, 8, 128), jnp.float32),
            pltpu.SemaphoreType.DMA,
            pltpu.SemaphoreType.DMA,
        ],
        compiler_params=pltpu.CompilerParams(use_tc_tiling_on_sc=False,
                                             needs_layout_passes=False),
    )


def _unview(x4):
    return (x4.transpose(0, 2, 1, 3).reshape(_T * _D, _B).T
            .reshape(_B, _T, 1, _D))


@jax.jit
def _run(h_t, r_t, t_t, ent, rel):
    gather = _make_gather()
    rel128 = _stage(rel.T, _NR)
    xr = gather(r_t, rel128)
    ent128 = _stage(ent.T, _NE)
    xh = gather(h_t, ent128)
    xt = gather(t_t, ent128)
    return _unview(xh), _unview(xr), _unview(xt)


def kernel(h_id, r_id, t_id, ent_transfer, rel_transfer):
    h_t = h_id.reshape(_B, _T).astype(jnp.int32).T
    r_t = r_id.reshape(_B, _T).astype(jnp.int32).T
    t_t = t_id.reshape(_B, _T).astype(jnp.int32).T
    return _run(h_t, r_t, t_t, ent_transfer, rel_transfer)
